# Initial kernel scaffold; baseline (speedup 1.0000x reference)
import jax
import jax.numpy as jnp
from jax.experimental import pallas as pl


def kernel(src_0, src_1, seg_0, seg_1, word_emb_0, pos_emb_0, seg_emb_0, gamma_0, beta_0, word_emb_1, pos_emb_1, seg_emb_1, gamma_1, beta_1):
    raise NotImplementedError("write your pallas kernel here")


# speedup vs baseline: 3.8597x; 3.8597x over previous
import functools

import jax
import jax.numpy as jnp
from jax import lax
from jax.experimental import pallas as pl
from jax.experimental.pallas import tpu as pltpu
from jax.experimental.pallas import tpu_sc as plsc

_V = 100000
_D = 64
_B = 1024
_S = 200
_NSEG = 3
_NTOK = _B * _S

_NC = 2
_NS = 16
_NW = _NC * _NS
_TPW = _NTOK // _NW
_CHUNK = 640
_NCHUNK = _TPW // _CHUNK
_GSUB = 128


@functools.cache
def _make_sc_gather():
    mesh = plsc.VectorSubcoreMesh(core_axis_name="c", subcore_axis_name="s")

    @functools.partial(
        pl.kernel,
        mesh=mesh,
        out_type=jax.ShapeDtypeStruct((_NTOK, _D), jnp.float32),
        scratch_types=[
            pltpu.VMEM((_CHUNK,), jnp.int32),
            pltpu.VMEM((_CHUNK, _D), jnp.float32),
            pltpu.SemaphoreType.DMA,
        ],
        compiler_params=pltpu.CompilerParams(use_tc_tiling_on_sc=False),
    )
    def gather_k(idx_hbm, table_hbm, out_hbm, idx_v, rows_v, sem):
        wid = lax.axis_index("s") * _NC + lax.axis_index("c")
        base0 = wid * _TPW
        for c in range(_NCHUNK):
            base = base0 + c * _CHUNK
            pltpu.sync_copy(idx_hbm.at[pl.ds(base, _CHUNK)], idx_v)
            copies = []
            for j in range(_CHUNK // _GSUB):
                copies.append(
                    pltpu.async_copy(
                        table_hbm.at[idx_v.at[pl.ds(j * _GSUB, _GSUB)]],
                        rows_v.at[pl.ds(j * _GSUB, _GSUB)],
                        sem,
                    )
                )
            for cp in copies:
                cp.wait()
            pltpu.sync_copy(rows_v, out_hbm.at[pl.ds(base, _CHUNK)])

    return gather_k


_BB = 8


def _tc_fuse_body(rows_ref, seg_ref, pos_ref, se_ref, gamma_ref, beta_ref, out_ref):
    x = rows_ref[...]
    g = seg_ref[...][:, :, None]
    e = x + pos_ref[...][None, :, :]
    se = se_ref[...]
    for k in range(_NSEG):
        e = e + jnp.where(g == k, se[k : k + 1][None], 0.0)
    mean = jnp.mean(e, axis=-1, keepdims=True)
    var = jnp.mean((e - mean) ** 2, axis=-1, keepdims=True)
    y = (e - mean) * lax.rsqrt(var + 1e-6)
    out_ref[...] = gamma_ref[...][None] * y + beta_ref[...][None]


def _tc_fuse(rows, seg, pos, se, gamma, beta):
    return pl.pallas_call(
        _tc_fuse_body,
        grid=(_B // _BB,),
        in_specs=[
            pl.BlockSpec((_BB, _S, _D), lambda i: (i, 0, 0)),
            pl.BlockSpec((_BB, _S), lambda i: (i, 0)),
            pl.BlockSpec((_S, _D), lambda i: (0, 0)),
            pl.BlockSpec((_NSEG, _D), lambda i: (0, 0)),
            pl.BlockSpec((1, _D), lambda i: (0, 0)),
            pl.BlockSpec((1, _D), lambda i: (0, 0)),
        ],
        out_specs=pl.BlockSpec((_BB, _S, _D), lambda i: (i, 0, 0)),
        out_shape=jax.ShapeDtypeStruct((_B, _S, _D), jnp.float32),
    )(rows, seg, pos, se, gamma, beta)


def _stream(src, seg, word_emb, pos_emb, seg_emb, gamma, beta):
    idx = src.reshape(-1).astype(jnp.int32)
    rows = _make_sc_gather()(idx, word_emb)
    return _tc_fuse(
        rows.reshape(_B, _S, _D),
        seg.astype(jnp.int32),
        pos_emb[:_S],
        seg_emb,
        gamma.reshape(1, _D),
        beta.reshape(1, _D),
    )


def kernel(src_0, src_1, seg_0, seg_1,
           word_emb_0, pos_emb_0, seg_emb_0, gamma_0, beta_0,
           word_emb_1, pos_emb_1, seg_emb_1, gamma_1, beta_1):
    out0 = _stream(src_0, seg_0, word_emb_0, pos_emb_0, seg_emb_0, gamma_0, beta_0)
    out1 = _stream(src_1, seg_1, word_emb_1, pos_emb_1, seg_emb_1, gamma_1, beta_1)
    return (out0, out1)
